# Initial kernel scaffold; baseline (speedup 1.0000x reference)
import jax
import jax.numpy as jnp
from jax.experimental import pallas as pl


def kernel(features_1, features_2, edge_index_1, edge_index_2, A_1, A_2, mapping, W1, b1, W2, b2, W3, b3, Wc, vc, Wm, vm, Wa, Wt, Wtb, bt, Wf1, bf1, Wf2, bf2, Wf3, bf3, Ws, bs):
    raise NotImplementedError("write your pallas kernel here")


# speedup vs baseline: 9.4869x; 9.4869x over previous
import functools

import jax
import jax.numpy as jnp
from jax import lax
from jax.experimental import pallas as pl
from jax.experimental.pallas import tpu as pltpu
from jax.experimental.pallas import tpu_sc as plsc

N = 1024
E = 32768
D_IN = 128
F1, F2, F3 = 128, 64, 32
K = 16
T = 16

_NS = 16
_ROWS = N // _NS
_CHUNK = 2048
_LANES = 16

_f32 = jnp.float32

def _sc_body(edges, zeros_blk, adj, acc, srcb, dstb):
    c = lax.axis_index("c")
    s = lax.axis_index("s")
    base = s * _ROWS
    pltpu.sync_copy(zeros_blk, acc)
    ones = jnp.ones((_LANES,), _f32)

    def chunk_body(t, carry):
        off = t * _CHUNK
        pltpu.sync_copy(edges.at[c, 0, pl.ds(off, _CHUNK)], srcb)
        pltpu.sync_copy(edges.at[c, 1, pl.ds(off, _CHUNK)], dstb)

        def step(i, carry2):
            src16 = srcb[pl.ds(i * _LANES, _LANES)]
            dst16 = dstb[pl.ds(i * _LANES, _LANES)]
            rel = dst16 - base
            mask = (rel >= 0) & (rel < _ROWS)
            flat = jnp.where(mask, rel * N + src16, 0)
            plsc.addupdate_scatter(acc, [flat], ones, mask=mask)
            return carry2

        return lax.fori_loop(0, _CHUNK // _LANES, step, carry)

    lax.fori_loop(0, E // _CHUNK, chunk_body, 0)
    pltpu.sync_copy(acc, adj.at[c, pl.ds(base * N, _ROWS * N)])


@functools.cache
def _sc_build_adj():
    mesh = plsc.VectorSubcoreMesh(core_axis_name="c", subcore_axis_name="s")
    return pl.kernel(
        _sc_body,
        mesh=mesh,
        compiler_params=pltpu.CompilerParams(needs_layout_passes=False),
        out_type=jax.ShapeDtypeStruct((2, N * N), _f32),
        scratch_types=[
            pltpu.VMEM((_ROWS * N,), _f32),
            pltpu.VMEM((_CHUNK,), jnp.int32),
            pltpu.VMEM((_CHUNK,), jnp.int32),
        ],
    )


def _mm(a, b, prec=None):
    return jnp.dot(a, b, preferred_element_type=_f32, precision=prec)


def _mm_nt(a, b, prec=None):
    return lax.dot_general(a, b, (((1,), (1,)), ((), ())),
                           preferred_element_type=_f32, precision=prec)


def _mm_tn(a, b, prec=None):
    return lax.dot_general(a, b, (((0,), (0,)), ((), ())),
                           preferred_element_type=_f32, precision=prec)


_HI = lax.Precision.HIGHEST


def _sigmoid(x):
    return 1.0 / (1.0 + jnp.exp(-x))


def _rsqrt(x):
    y = lax.rsqrt(x)
    return y * (1.5 - 0.5 * x * y * y)


def _tc_body(vc_s, vm_s, bs_s,
             A1r, A2r, x1r, x2r,
             W1r, b1r, W2r, b2r, W3r, b3r,
             Wcr, Wmr, War, Wttr, btr, WtbAr, WtbBr,
             Wf1r, bf1r, Wf2r, bf2r, Wf3r, bf3r, Wsr,
             mapm_ref, score_ref):
    rows = lax.broadcasted_iota(jnp.int32, (N, N), 0)
    cols = lax.broadcasted_iota(jnp.int32, (N, N), 1)
    eye = jnp.where(rows == cols, 1.0, 0.0)
    ones_row = jnp.ones((1, N), _f32)

    def gcn(Ar, xr):
        A = Ar[...]
        deg_col = jnp.sum(A, axis=1, keepdims=True) + 1.0
        deg_row = _mm_nt(ones_row, A, _HI) + 1.0
        dinv_col = _rsqrt(deg_col)
        dinv_row = _rsqrt(deg_row)
        S = (A + eye) * (dinv_col * dinv_row)
        h = jnp.maximum(_mm(S, _mm(xr[...], W1r[...]), _HI) + b1r[...], 0.0)
        h = jnp.maximum(_mm(S, _mm(h, W2r[...]), _HI) + b2r[...], 0.0)
        return _mm(S, _mm(h, W3r[...]), _HI) + b3r[...]

    af1 = gcn(A1r, x1r)
    af2 = gcn(A2r, x2r)

    cost = jnp.zeros((N, N), _f32)
    mapm = jnp.zeros((N, N), _f32)
    bf = jnp.bfloat16
    for k in range(K):
        mc = _mm_nt(_mm(af1, Wcr[k]), af2)
        cost = cost + vc_s[0, k] * jnp.maximum(mc, 0.0)
        mk = _mm_nt(_mm(af1, Wmr[k]), af2)
        rk = jnp.maximum(mk, 0.0).astype(bf).astype(_f32)
        vk = vm_s[0, k].astype(bf).astype(_f32)
        mapm = mapm + vk * rk
    mapm_ref[...] = mapm

    rowmax = jnp.max(mapm, axis=1, keepdims=True)
    ex = jnp.exp(mapm - rowmax)
    soft = ex / jnp.sum(ex, axis=1, keepdims=True)
    tot_col = jnp.sum(soft * cost, axis=1, keepdims=True)
    total11 = jnp.sum(tot_col, axis=0, keepdims=True)

    def attention(af):
        g = jnp.tanh(jnp.mean(_mm(af, War[...]), axis=0, keepdims=True))
        sgate = _sigmoid(_mm_nt(af, g))
        return _mm_tn(sgate, af)

    p1 = attention(af1)
    p2 = attention(af2)

    G = _mm_tn(p1, p2)
    colid = lax.broadcasted_iota(jnp.int32, (1, T), 1)
    sc_row = jnp.zeros((1, T), _f32)
    for t in range(T):
        st = jnp.sum(G * Wttr[t])
        sc_row = sc_row + jnp.where(colid == t, st, 0.0)
    blk_row = _mm(p1, WtbAr[...]) + _mm(p2, WtbBr[...])
    sv = jnp.maximum(sc_row + blk_row + btr[...], 0.0)
    sv = jnp.maximum(_mm(sv, Wf1r[...]) + bf1r[...], 0.0)
    sv = jnp.maximum(_mm(sv, Wf2r[...]) + bf2r[...], 0.0)
    sv = jnp.maximum(_mm(sv, Wf3r[...]) + bf3r[...], 0.0)
    bias11 = _mm(sv, Wsr[...]) + bs_s[0, 0]
    score_ref[...] = _sigmoid(total11 + bias11)


def _tc_call(vc2, vm2, bs2, A1, A2, x1, x2, *dense_ops):
    n_smem = 3
    n_vmem = 4 + len(dense_ops)
    in_specs = ([pl.BlockSpec(memory_space=pltpu.SMEM)] * n_smem
                + [pl.BlockSpec(memory_space=pltpu.VMEM)] * n_vmem)
    return pl.pallas_call(
        _tc_body,
        out_shape=(jax.ShapeDtypeStruct((N, N), _f32),
                   jax.ShapeDtypeStruct((1, 1), _f32)),
        in_specs=in_specs,
        out_specs=(pl.BlockSpec(memory_space=pltpu.VMEM),
                   pl.BlockSpec(memory_space=pltpu.VMEM)),
    )(vc2, vm2, bs2, A1, A2, x1, x2, *dense_ops)


def kernel(features_1, features_2, edge_index_1, edge_index_2, A_1, A_2,
           mapping, W1, b1, W2, b2, W3, b3, Wc, vc, Wm, vm, Wa, Wt, Wtb, bt,
           Wf1, bf1, Wf2, bf2, Wf3, bf3, Ws, bs):
    edges = jnp.stack([edge_index_1.astype(jnp.int32),
                       edge_index_2.astype(jnp.int32)])
    zeros_blk = jnp.zeros((_ROWS * N,), _f32)
    adj = _sc_build_adj()(edges, zeros_blk).reshape(2, N, N)

    mapm, score11 = _tc_call(
        vc.reshape(1, K), vm.reshape(1, K), bs.reshape(1, 1),
        adj[0], adj[1], features_1, features_2,
        W1, b1.reshape(1, F1), W2, b2.reshape(1, F2), W3, b3.reshape(1, F3),
        Wc, Wm, Wa,
        jnp.transpose(Wt, (2, 0, 1)),
        bt.reshape(1, T),
        jnp.transpose(Wtb[:, :F3]),
        jnp.transpose(Wtb[:, F3:]),
        Wf1, bf1.reshape(1, -1), Wf2, bf2.reshape(1, -1),
        Wf3, bf3.reshape(1, -1), Ws)
    return (mapm, score11.reshape(-1))


# speedup vs baseline: 10.7508x; 1.1332x over previous
import functools

import jax
import jax.numpy as jnp
from jax import lax
from jax.experimental import pallas as pl
from jax.experimental.pallas import tpu as pltpu
from jax.experimental.pallas import tpu_sc as plsc

N = 1024
E = 32768
D_IN = 128
F1, F2, F3 = 128, 64, 32
K = 16
T = 16

_NS = 16
_ROWS = N // _NS
_CHUNK = 16384
_LANES = 16

_f32 = jnp.float32

def _sc_body(edges, adj, acc, srcb, dstb):
    c = lax.axis_index("c")
    s = lax.axis_index("s")
    base = s * _ROWS
    zeros16 = jnp.zeros((_LANES,), _f32)
    ones = jnp.ones((_LANES,), _f32)

    def zbody(j, carry):
        acc[pl.ds(j * _LANES, _LANES)] = zeros16
        return carry

    lax.fori_loop(0, _ROWS * N // _LANES, zbody, 0, unroll=16)

    for t in range(E // _CHUNK):
        off = t * _CHUNK
        pltpu.sync_copy(edges.at[c, 0, pl.ds(off, _CHUNK)], srcb)
        pltpu.sync_copy(edges.at[c, 1, pl.ds(off, _CHUNK)], dstb)

        def step(i, carry2):
            src16 = srcb[pl.ds(i * _LANES, _LANES)]
            dst16 = dstb[pl.ds(i * _LANES, _LANES)]
            rel = dst16 - base
            mask = plsc.bitcast(rel, jnp.uint32) < jnp.uint32(_ROWS)
            flat = (rel << 10) | src16
            plsc.addupdate_scatter(acc, [flat], ones, mask=mask)
            return carry2

        lax.fori_loop(0, _CHUNK // _LANES, step, 0, unroll=8)

    pltpu.sync_copy(acc, adj.at[c, pl.ds(base * N, _ROWS * N)])


@functools.cache
def _sc_build_adj():
    mesh = plsc.VectorSubcoreMesh(core_axis_name="c", subcore_axis_name="s")
    return pl.kernel(
        _sc_body,
        mesh=mesh,
        compiler_params=pltpu.CompilerParams(needs_layout_passes=False),
        out_type=jax.ShapeDtypeStruct((2, N * N), _f32),
        scratch_types=[
            pltpu.VMEM((_ROWS * N,), _f32),
            pltpu.VMEM((_CHUNK,), jnp.int32),
            pltpu.VMEM((_CHUNK,), jnp.int32),
        ],
    )


def _mm(a, b, prec=None):
    return jnp.dot(a, b, preferred_element_type=_f32, precision=prec)


def _mm_nt(a, b, prec=None):
    return lax.dot_general(a, b, (((1,), (1,)), ((), ())),
                           preferred_element_type=_f32, precision=prec)


def _mm_tn(a, b, prec=None):
    return lax.dot_general(a, b, (((0,), (0,)), ((), ())),
                           preferred_element_type=_f32, precision=prec)


_HI = lax.Precision.HIGHEST


def _sigmoid(x):
    return 1.0 / (1.0 + jnp.exp(-x))


def _rsqrt(x):
    y = lax.rsqrt(x)
    return y * (1.5 - 0.5 * x * y * y)


def _tc_body(vc_s, vm_s, bs_s,
             A1r, A2r, x1r, x2r,
             W1r, b1r, W2r, b2r, W3r, b3r,
             Wcr, Wmr, War, Wttr, btr, WtbAr, WtbBr,
             Wf1r, bf1r, Wf2r, bf2r, Wf3r, bf3r, Wsr,
             mapm_ref, score_ref):
    rows = lax.broadcasted_iota(jnp.int32, (N, N), 0)
    cols = lax.broadcasted_iota(jnp.int32, (N, N), 1)
    eye = jnp.where(rows == cols, 1.0, 0.0)
    ones_row = jnp.ones((1, N), _f32)

    def gcn(Ar, xr):
        A = Ar[...]
        deg_col = jnp.sum(A, axis=1, keepdims=True) + 1.0
        deg_row = _mm_nt(ones_row, A, _HI) + 1.0
        dinv_col = _rsqrt(deg_col)
        dinv_row = _rsqrt(deg_row)
        S = (A + eye) * (dinv_col * dinv_row)
        h = jnp.maximum(_mm(S, _mm(xr[...], W1r[...]), _HI) + b1r[...], 0.0)
        h = jnp.maximum(_mm(S, _mm(h, W2r[...]), _HI) + b2r[...], 0.0)
        return _mm(S, _mm(h, W3r[...]), _HI) + b3r[...]

    af1 = gcn(A1r, x1r)
    af2 = gcn(A2r, x2r)

    cost = jnp.zeros((N, N), _f32)
    mapm = jnp.zeros((N, N), _f32)
    bf = jnp.bfloat16
    for k in range(K):
        mc = _mm_nt(_mm(af1, Wcr[k]), af2)
        cost = cost + vc_s[0, k] * jnp.maximum(mc, 0.0)
        mk = _mm_nt(_mm(af1, Wmr[k]), af2)
        rk = jnp.maximum(mk, 0.0).astype(bf).astype(_f32)
        vk = vm_s[0, k].astype(bf).astype(_f32)
        mapm = mapm + vk * rk
    mapm_ref[...] = mapm

    rowmax = jnp.max(mapm, axis=1, keepdims=True)
    ex = jnp.exp(mapm - rowmax)
    soft = ex / jnp.sum(ex, axis=1, keepdims=True)
    tot_col = jnp.sum(soft * cost, axis=1, keepdims=True)
    total11 = jnp.sum(tot_col, axis=0, keepdims=True)

    def attention(af):
        g = jnp.tanh(jnp.mean(_mm(af, War[...]), axis=0, keepdims=True))
        sgate = _sigmoid(_mm_nt(af, g))
        return _mm_tn(sgate, af)

    p1 = attention(af1)
    p2 = attention(af2)

    G = _mm_tn(p1, p2)
    colid = lax.broadcasted_iota(jnp.int32, (1, T), 1)
    sc_row = jnp.zeros((1, T), _f32)
    for t in range(T):
        st = jnp.sum(G * Wttr[t])
        sc_row = sc_row + jnp.where(colid == t, st, 0.0)
    blk_row = _mm(p1, WtbAr[...]) + _mm(p2, WtbBr[...])
    sv = jnp.maximum(sc_row + blk_row + btr[...], 0.0)
    sv = jnp.maximum(_mm(sv, Wf1r[...]) + bf1r[...], 0.0)
    sv = jnp.maximum(_mm(sv, Wf2r[...]) + bf2r[...], 0.0)
    sv = jnp.maximum(_mm(sv, Wf3r[...]) + bf3r[...], 0.0)
    bias11 = _mm(sv, Wsr[...]) + bs_s[0, 0]
    score_ref[...] = _sigmoid(total11 + bias11)


def _tc_call(vc2, vm2, bs2, A1, A2, x1, x2, *dense_ops):
    n_smem = 3
    n_vmem = 4 + len(dense_ops)
    in_specs = ([pl.BlockSpec(memory_space=pltpu.SMEM)] * n_smem
                + [pl.BlockSpec(memory_space=pltpu.VMEM)] * n_vmem)
    return pl.pallas_call(
        _tc_body,
        out_shape=(jax.ShapeDtypeStruct((N, N), _f32),
                   jax.ShapeDtypeStruct((1, 1), _f32)),
        in_specs=in_specs,
        out_specs=(pl.BlockSpec(memory_space=pltpu.VMEM),
                   pl.BlockSpec(memory_space=pltpu.VMEM)),
    )(vc2, vm2, bs2, A1, A2, x1, x2, *dense_ops)


def kernel(features_1, features_2, edge_index_1, edge_index_2, A_1, A_2,
           mapping, W1, b1, W2, b2, W3, b3, Wc, vc, Wm, vm, Wa, Wt, Wtb, bt,
           Wf1, bf1, Wf2, bf2, Wf3, bf3, Ws, bs):
    edges = jnp.stack([edge_index_1.astype(jnp.int32),
                       edge_index_2.astype(jnp.int32)])
    adj = _sc_build_adj()(edges).reshape(2, N, N)

    mapm, score11 = _tc_call(
        vc.reshape(1, K), vm.reshape(1, K), bs.reshape(1, 1),
        adj[0], adj[1], features_1, features_2,
        W1, b1.reshape(1, F1), W2, b2.reshape(1, F2), W3, b3.reshape(1, F3),
        Wc, Wm, Wa,
        jnp.transpose(Wt, (2, 0, 1)),
        bt.reshape(1, T),
        jnp.transpose(Wtb[:, :F3]),
        jnp.transpose(Wtb[:, F3:]),
        Wf1, bf1.reshape(1, -1), Wf2, bf2.reshape(1, -1),
        Wf3, bf3.reshape(1, -1), Ws)
    return (mapm, score11.reshape(-1))


# speedup vs baseline: 13.6414x; 1.2689x over previous
import functools

import jax
import jax.numpy as jnp
from jax import lax
from jax.experimental import pallas as pl
from jax.experimental.pallas import tpu as pltpu
from jax.experimental.pallas import tpu_sc as plsc

N = 1024
E = 32768
D_IN = 128
F1, F2, F3 = 128, 64, 32
K = 16
T = 16

_NS = 16
_ROWS = N // _NS
_CHUNK = 16384
_LANES = 16

_f32 = jnp.float32

def _sc_body(edges, adj, acc, srcb, dstb):
    c = lax.axis_index("c")
    s = lax.axis_index("s")
    base = s * _ROWS
    zeros16 = jnp.zeros((_LANES,), _f32)
    ones = jnp.ones((_LANES,), _f32)

    def zbody(j, carry):
        acc[j >> 6, pl.ds((j & 63) * _LANES, _LANES)] = zeros16
        return carry

    lax.fori_loop(0, _ROWS * N // _LANES, zbody, 0, unroll=16)

    for t in range(E // _CHUNK):
        off = t * _CHUNK
        pltpu.sync_copy(edges.at[c, 0, pl.ds(off, _CHUNK)], srcb)
        pltpu.sync_copy(edges.at[c, 1, pl.ds(off, _CHUNK)], dstb)

        def step(i, carry2):
            src16 = srcb[pl.ds(i * _LANES, _LANES)]
            dst16 = dstb[pl.ds(i * _LANES, _LANES)]
            rel = dst16 - base
            mask = plsc.bitcast(rel, jnp.uint32) < jnp.uint32(_ROWS)
            col = jnp.where(mask, src16, 0)
            plsc.addupdate_scatter(acc, [rel, col], ones, mask=mask)
            return carry2

        lax.fori_loop(0, _CHUNK // _LANES, step, 0, unroll=8)

    pltpu.sync_copy(acc, adj.at[c, pl.ds(base, _ROWS)])


@functools.cache
def _sc_build_adj():
    mesh = plsc.VectorSubcoreMesh(core_axis_name="c", subcore_axis_name="s")
    return pl.kernel(
        _sc_body,
        mesh=mesh,
        compiler_params=pltpu.CompilerParams(needs_layout_passes=False),
        out_type=jax.ShapeDtypeStruct((2, N, N), _f32),
        scratch_types=[
            pltpu.VMEM((_ROWS, N), _f32),
            pltpu.VMEM((_CHUNK,), jnp.int32),
            pltpu.VMEM((_CHUNK,), jnp.int32),
        ],
    )


def _mm(a, b, prec=None):
    return jnp.dot(a, b, preferred_element_type=_f32, precision=prec)


def _mm_nt(a, b, prec=None):
    return lax.dot_general(a, b, (((1,), (1,)), ((), ())),
                           preferred_element_type=_f32, precision=prec)


def _mm_tn(a, b, prec=None):
    return lax.dot_general(a, b, (((0,), (0,)), ((), ())),
                           preferred_element_type=_f32, precision=prec)


_HI = lax.Precision.HIGHEST


def _sigmoid(x):
    return 1.0 / (1.0 + jnp.exp(-x))


def _rsqrt(x):
    y = lax.rsqrt(x)
    return y * (1.5 - 0.5 * x * y * y)


def _tc_body(vc_s, vm_s, bs_s,
             A1r, A2r, x1r, x2r,
             W1r, b1r, W2r, b2r, W3r, b3r,
             Wcr, Wmr, War, Wttr, btr, WtbAr, WtbBr,
             Wf1r, bf1r, Wf2r, bf2r, Wf3r, bf3r, Wsr,
             mapm_ref, score_ref):
    rows = lax.broadcasted_iota(jnp.int32, (N, N), 0)
    cols = lax.broadcasted_iota(jnp.int32, (N, N), 1)
    eye = jnp.where(rows == cols, 1.0, 0.0)
    ones_row = jnp.ones((1, N), _f32)

    def gcn(Ar, xr):
        A = Ar[...]
        deg_col = jnp.sum(A, axis=1, keepdims=True) + 1.0
        deg_row = _mm_nt(ones_row, A, _HI) + 1.0
        dinv_col = _rsqrt(deg_col)
        dinv_row = _rsqrt(deg_row)
        S = (A + eye) * (dinv_col * dinv_row)
        h = jnp.maximum(_mm(S, _mm(xr[...], W1r[...]), _HI) + b1r[...], 0.0)
        h = jnp.maximum(_mm(S, _mm(h, W2r[...]), _HI) + b2r[...], 0.0)
        return _mm(S, _mm(h, W3r[...]), _HI) + b3r[...]

    af1 = gcn(A1r, x1r)
    af2 = gcn(A2r, x2r)

    cost = jnp.zeros((N, N), _f32)
    mapm = jnp.zeros((N, N), _f32)
    bf = jnp.bfloat16
    for k in range(K):
        mc = _mm_nt(_mm(af1, Wcr[k]), af2)
        cost = cost + vc_s[0, k] * jnp.maximum(mc, 0.0)
        mk = _mm_nt(_mm(af1, Wmr[k]), af2)
        rk = jnp.maximum(mk, 0.0).astype(bf).astype(_f32)
        vk = vm_s[0, k].astype(bf).astype(_f32)
        mapm = mapm + vk * rk
    mapm_ref[...] = mapm

    rowmax = jnp.max(mapm, axis=1, keepdims=True)
    ex = jnp.exp(mapm - rowmax)
    soft = ex / jnp.sum(ex, axis=1, keepdims=True)
    tot_col = jnp.sum(soft * cost, axis=1, keepdims=True)
    total11 = jnp.sum(tot_col, axis=0, keepdims=True)

    def attention(af):
        g = jnp.tanh(jnp.mean(_mm(af, War[...]), axis=0, keepdims=True))
        sgate = _sigmoid(_mm_nt(af, g))
        return _mm_tn(sgate, af)

    p1 = attention(af1)
    p2 = attention(af2)

    G = _mm_tn(p1, p2)
    colid = lax.broadcasted_iota(jnp.int32, (1, T), 1)
    sc_row = jnp.zeros((1, T), _f32)
    for t in range(T):
        st = jnp.sum(G * Wttr[t])
        sc_row = sc_row + jnp.where(colid == t, st, 0.0)
    blk_row = _mm(p1, WtbAr[...]) + _mm(p2, WtbBr[...])
    sv = jnp.maximum(sc_row + blk_row + btr[...], 0.0)
    sv = jnp.maximum(_mm(sv, Wf1r[...]) + bf1r[...], 0.0)
    sv = jnp.maximum(_mm(sv, Wf2r[...]) + bf2r[...], 0.0)
    sv = jnp.maximum(_mm(sv, Wf3r[...]) + bf3r[...], 0.0)
    bias11 = _mm(sv, Wsr[...]) + bs_s[0, 0]
    score_ref[...] = _sigmoid(total11 + bias11)


def _tc_call(vc2, vm2, bs2, A1, A2, x1, x2, *dense_ops):
    n_smem = 3
    n_vmem = 4 + len(dense_ops)
    in_specs = ([pl.BlockSpec(memory_space=pltpu.SMEM)] * n_smem
                + [pl.BlockSpec(memory_space=pltpu.VMEM)] * n_vmem)
    return pl.pallas_call(
        _tc_body,
        out_shape=(jax.ShapeDtypeStruct((N, N), _f32),
                   jax.ShapeDtypeStruct((1, 1), _f32)),
        in_specs=in_specs,
        out_specs=(pl.BlockSpec(memory_space=pltpu.VMEM),
                   pl.BlockSpec(memory_space=pltpu.VMEM)),
    )(vc2, vm2, bs2, A1, A2, x1, x2, *dense_ops)


def kernel(features_1, features_2, edge_index_1, edge_index_2, A_1, A_2,
           mapping, W1, b1, W2, b2, W3, b3, Wc, vc, Wm, vm, Wa, Wt, Wtb, bt,
           Wf1, bf1, Wf2, bf2, Wf3, bf3, Ws, bs):
    edges = jnp.stack([edge_index_1.astype(jnp.int32),
                       edge_index_2.astype(jnp.int32)])
    adj = _sc_build_adj()(edges)

    mapm, score11 = _tc_call(
        vc.reshape(1, K), vm.reshape(1, K), bs.reshape(1, 1),
        adj[0], adj[1], features_1, features_2,
        W1, b1.reshape(1, F1), W2, b2.reshape(1, F2), W3, b3.reshape(1, F3),
        Wc, Wm, Wa,
        jnp.transpose(Wt, (2, 0, 1)),
        bt.reshape(1, T),
        jnp.transpose(Wtb[:, :F3]),
        jnp.transpose(Wtb[:, F3:]),
        Wf1, bf1.reshape(1, -1), Wf2, bf2.reshape(1, -1),
        Wf3, bf3.reshape(1, -1), Ws)
    return (mapm, score11.reshape(-1))
